# Initial kernel scaffold; baseline (speedup 1.0000x reference)
import jax
import jax.numpy as jnp
from jax.experimental import pallas as pl


def kernel(x, pos_table):
    raise NotImplementedError("write your pallas kernel here")


# speedup vs baseline: 2.8581x; 2.8581x over previous
import jax
import jax.numpy as jnp
from jax.experimental import pallas as pl

_BP = 512


def _add_body(x_ref, pos_ref, out_ref):
    out_ref[...] = x_ref[...] + pos_ref[...][None]


def kernel(x, pos_table):
    B, P, D = x.shape
    grid = (P // _BP, B)
    return pl.pallas_call(
        _add_body,
        grid=grid,
        in_specs=[
            pl.BlockSpec((1, _BP, D), lambda p, b: (b, p, 0)),
            pl.BlockSpec((_BP, D), lambda p, b: (p, 0)),
        ],
        out_specs=pl.BlockSpec((1, _BP, D), lambda p, b: (b, p, 0)),
        out_shape=jax.ShapeDtypeStruct((B, P, D), x.dtype),
    )(x, pos_table)
